# baseline (speedup 1.0000x reference)
import jax
import jax.numpy as jnp
from jax import lax
from jax.experimental import pallas as pl
from jax.experimental.pallas import tpu as pltpu
from jax.experimental.pallas import tpu_sc as plsc

_N = 100000
_D = 128
_NW = 32
_EPW = _N * _D // _NW
_CHE = 16000
_CHR = _CHE // _D
_NCH = _EPW // _CHE


def _sc_body(x1_hbm, x2_hbm, w_hbm, b_hbm, y1_hbm, y2_hbm,
             xbuf, ybuf, wbuf, bbuf):
    wid = lax.axis_index("s") * 2 + lax.axis_index("c")
    base = wid * _EPW
    pltpu.sync_copy(w_hbm, wbuf)
    pltpu.sync_copy(b_hbm, bbuf)
    for inp, (x_hbm, y_hbm) in enumerate(((x1_hbm, y1_hbm), (x2_hbm, y2_hbm))):
        wvs = [wbuf[inp, pl.ds(g * 16, 16)] for g in range(8)]
        bvs = [bbuf[inp, pl.ds(g * 16, 16)] for g in range(8)]

        def chunk_body(k, carry):
            e0 = base + k * _CHE
            pltpu.sync_copy(x_hbm.at[pl.ds(e0, _CHE)], xbuf)

            def row_body(r, c2):
                for g in range(8):
                    o = r * _D + g * 16
                    v = xbuf[pl.ds(o, 16)]
                    ybuf[pl.ds(o, 16)] = jnp.maximum(v, 0.0) * wvs[g] + bvs[g]
                return c2

            lax.fori_loop(0, _CHR, row_body, 0)
            pltpu.sync_copy(ybuf, y_hbm.at[pl.ds(e0, _CHE)])
            return carry

        lax.fori_loop(0, _NCH, chunk_body, 0)


def kernel(x1, x2, W, b):
    n, d = x1.shape
    wstack = W.reshape(2, d)
    bstack = b.reshape(2, d)
    x1f = x1.reshape(n * d)
    x2f = x2.reshape(n * d)

    sc_fn = pl.kernel(
        _sc_body,
        out_type=[jax.ShapeDtypeStruct((n * d,), x1.dtype)] * 2,
        mesh=plsc.VectorSubcoreMesh(core_axis_name="c", subcore_axis_name="s"),
        scratch_types=[
            pltpu.VMEM((_CHE,), jnp.float32),
            pltpu.VMEM((_CHE,), jnp.float32),
            pltpu.VMEM((2, d), jnp.float32),
            pltpu.VMEM((2, d), jnp.float32),
        ],
    )
    y1f, y2f = sc_fn(x1f, x2f, wstack, bstack)
    return (y1f.reshape(n, d), y2f.reshape(n, d))


# speedup vs baseline: 1.4255x; 1.4255x over previous
import jax
import jax.numpy as jnp
from jax import lax
from jax.experimental import pallas as pl
from jax.experimental.pallas import tpu as pltpu
from jax.experimental.pallas import tpu_sc as plsc

_N = 100000
_D = 128
_NW = 32
_EPW = _N * _D // _NW
_CHE = 16000
_CHR = _CHE // _D
_NCH = _EPW // _CHE


def _tc_kernel(x_ref, w_ref, b_ref, y_ref):
    y_ref[...] = jnp.maximum(x_ref[...], 0.0) * w_ref[...] + b_ref[...]


def _sc_body(x_hbm, w_hbm, b_hbm, y_hbm, xbuf, ybuf, wbuf, bbuf):
    wid = lax.axis_index("s") * 2 + lax.axis_index("c")
    base = wid * _EPW
    pltpu.sync_copy(w_hbm, wbuf)
    pltpu.sync_copy(b_hbm, bbuf)
    wvs = [wbuf[0, pl.ds(g * 16, 16)] for g in range(8)]
    bvs = [bbuf[0, pl.ds(g * 16, 16)] for g in range(8)]

    def chunk_body(k, carry):
        e0 = base + k * _CHE
        pltpu.sync_copy(x_hbm.at[pl.ds(e0, _CHE)], xbuf)

        def row_body(r, c2):
            for g in range(8):
                o = r * _D + g * 16
                v = xbuf[pl.ds(o, 16)]
                ybuf[pl.ds(o, 16)] = jnp.maximum(v, 0.0) * wvs[g] + bvs[g]
            return c2

        lax.fori_loop(0, _CHR, row_body, 0)
        pltpu.sync_copy(ybuf, y_hbm.at[pl.ds(e0, _CHE)])
        return carry

    lax.fori_loop(0, _NCH, chunk_body, 0)


def kernel(x1, x2, W, b):
    n, d = x1.shape
    w1 = W[:d].reshape(1, d)
    b1 = b[:d].reshape(1, d)
    w2 = W[d:].reshape(1, d)
    b2 = b[d:].reshape(1, d)

    sc_fn = pl.kernel(
        _sc_body,
        out_type=jax.ShapeDtypeStruct((n * d,), x2.dtype),
        mesh=plsc.VectorSubcoreMesh(core_axis_name="c", subcore_axis_name="s"),
        scratch_types=[
            pltpu.VMEM((_CHE,), jnp.float32),
            pltpu.VMEM((_CHE,), jnp.float32),
            pltpu.VMEM((1, d), jnp.float32),
            pltpu.VMEM((1, d), jnp.float32),
        ],
    )
    y2 = sc_fn(x2.reshape(n * d), w2, b2).reshape(n, d)

    block_rows = 10000
    bs_x = pl.BlockSpec((block_rows, d), lambda i: (i, 0))
    bs_w = pl.BlockSpec((1, d), lambda i: (0, 0))
    y1 = pl.pallas_call(
        _tc_kernel,
        grid=(n // block_rows,),
        in_specs=[bs_x, bs_w, bs_w],
        out_specs=bs_x,
        out_shape=jax.ShapeDtypeStruct((n, d), x1.dtype),
    )(x1, w1, b1)
    return (y1, y2)


# speedup vs baseline: 1.8075x; 1.2679x over previous
import jax
import jax.numpy as jnp
from jax import lax
from jax.experimental import pallas as pl
from jax.experimental.pallas import tpu as pltpu
from jax.experimental.pallas import tpu_sc as plsc

_N = 100000
_D = 128
_NW = 32
_EPW = _N * _D // _NW
_CHE = 16000
_CHR = _CHE // _D
_NCH = _EPW // _CHE


def _tc_kernel(x_ref, w_ref, b_ref, y_ref):
    y_ref[...] = jnp.maximum(x_ref[...], 0.0) * w_ref[...] + b_ref[...]


def _sc_body(x_hbm, w_hbm, b_hbm, y_hbm,
             xb0, xb1, yb0, yb1, wbuf, bbuf, si0, si1, so0, so1):
    wid = lax.axis_index("s") * 2 + lax.axis_index("c")
    base = wid * _EPW
    pltpu.sync_copy(w_hbm, wbuf)
    pltpu.sync_copy(b_hbm, bbuf)
    wvs = [wbuf[0, pl.ds(g * 16, 16)] for g in range(8)]
    bvs = [bbuf[0, pl.ds(g * 16, 16)] for g in range(8)]

    xb = (xb0, xb1)
    yb = (yb0, yb1)
    si = (si0, si1)
    so = (so0, so1)

    def load(k, buf, sem):
        return pltpu.async_copy(x_hbm.at[pl.ds(base + k * _CHE, _CHE)], buf, sem)

    def store(k, buf, sem):
        return pltpu.async_copy(buf, y_hbm.at[pl.ds(base + k * _CHE, _CHE)], sem)

    loads = [None] * _NCH
    stores = [None] * _NCH
    loads[0] = load(0, xb[0], si[0])
    for k in range(_NCH):
        c = k & 1
        if k + 1 < _NCH:
            loads[k + 1] = load(k + 1, xb[1 - c], si[1 - c])
        loads[k].wait()
        if k >= 2:
            stores[k - 2].wait()
        xbuf = xb[c]
        ybuf = yb[c]

        def row_body(r, c2):
            for g in range(8):
                o = r * _D + g * 16
                v = xbuf[pl.ds(o, 16)]
                ybuf[pl.ds(o, 16)] = jnp.maximum(v, 0.0) * wvs[g] + bvs[g]
            return c2

        lax.fori_loop(0, _CHR, row_body, 0)
        stores[k] = store(k, yb[c], so[c])
    stores[_NCH - 2].wait()
    stores[_NCH - 1].wait()


def kernel(x1, x2, W, b):
    n, d = x1.shape
    w1 = W[:d].reshape(1, d)
    b1 = b[:d].reshape(1, d)
    w2 = W[d:].reshape(1, d)
    b2 = b[d:].reshape(1, d)

    sc_fn = pl.kernel(
        _sc_body,
        out_type=jax.ShapeDtypeStruct((n * d,), x2.dtype),
        mesh=plsc.VectorSubcoreMesh(core_axis_name="c", subcore_axis_name="s"),
        scratch_types=[
            pltpu.VMEM((_CHE,), jnp.float32),
            pltpu.VMEM((_CHE,), jnp.float32),
            pltpu.VMEM((_CHE,), jnp.float32),
            pltpu.VMEM((_CHE,), jnp.float32),
            pltpu.VMEM((1, d), jnp.float32),
            pltpu.VMEM((1, d), jnp.float32),
            pltpu.SemaphoreType.DMA,
            pltpu.SemaphoreType.DMA,
            pltpu.SemaphoreType.DMA,
            pltpu.SemaphoreType.DMA,
        ],
    )
    y2 = sc_fn(x2.reshape(n * d), w2, b2).reshape(n, d)

    block_rows = 10000
    bs_x = pl.BlockSpec((block_rows, d), lambda i: (i, 0))
    bs_w = pl.BlockSpec((1, d), lambda i: (0, 0))
    y1 = pl.pallas_call(
        _tc_kernel,
        grid=(n // block_rows,),
        in_specs=[bs_x, bs_w, bs_w],
        out_specs=bs_x,
        out_shape=jax.ShapeDtypeStruct((n, d), x1.dtype),
    )(x1, w1, b1)
    return (y1, y2)


# speedup vs baseline: 2.4327x; 1.3459x over previous
import jax
import jax.numpy as jnp
from jax.experimental import pallas as pl


def _ew_kernel(x_ref, w_ref, b_ref, y_ref):
    y_ref[...] = jnp.maximum(x_ref[...], 0.0) * w_ref[...] + b_ref[...]


def _stream(x, w, b, block_rows):
    n, d = x.shape
    bs_x = pl.BlockSpec((block_rows, d), lambda i: (i, 0))
    bs_w = pl.BlockSpec((1, d), lambda i: (0, 0))
    return pl.pallas_call(
        _ew_kernel,
        grid=(n // block_rows,),
        in_specs=[bs_x, bs_w, bs_w],
        out_specs=bs_x,
        out_shape=jax.ShapeDtypeStruct((n, d), x.dtype),
    )(x, w.reshape(1, d), b.reshape(1, d))


def kernel(x1, x2, W, b):
    n, d = x1.shape
    y1 = _stream(x1, W[:d], b[:d], 25000)
    y2 = _stream(x2, W[d:], b[d:], 25000)
    return (y1, y2)
